# Initial kernel scaffold; baseline (speedup 1.0000x reference)
import jax
import jax.numpy as jnp
from jax.experimental import pallas as pl


def kernel(inputs, mapping_array):
    raise NotImplementedError("write your pallas kernel here")


# speedup vs baseline: 2.9329x; 2.9329x over previous
import functools

import jax
import jax.numpy as jnp
from jax import lax
from jax.experimental import pallas as pl
from jax.experimental.pallas import tpu as pltpu
from jax.experimental.pallas import tpu_sc as plsc

NUM_BITS = 6
N = 64


def _pack_body(bits_ref, w_ref, idx_ref):
    acc = jnp.dot(bits_ref[...], w_ref[...], preferred_element_type=jnp.float32)
    idx_ref[...] = acc.astype(jnp.int32)


def _pack_indices(bits, w, block_rows):
    batch, feat = bits.shape
    num_ccsk = feat // NUM_BITS
    grid = (batch // block_rows,)
    return pl.pallas_call(
        _pack_body,
        grid=grid,
        in_specs=[
            pl.BlockSpec((block_rows, feat), lambda i: (i, 0)),
            pl.BlockSpec((feat, num_ccsk), lambda i: (0, 0)),
        ],
        out_specs=pl.BlockSpec((block_rows, num_ccsk), lambda i: (i, 0)),
        out_shape=jax.ShapeDtypeStruct((batch, num_ccsk), jnp.int32),
    )(bits, w)


def _sc_info():
    try:
        info = plsc.get_sparse_core_info()
        return info.num_cores, info.num_subcores
    except Exception:
        return 2, 16


def _gather_rows(table, idx_flat, total_rows):
    nc, ns = _sc_info()
    nw = nc * ns
    b_per_w = total_rows // nw
    chunk = min(1024, b_per_w)
    sub = 128
    n_sub = chunk // sub
    n_chunks = b_per_w // chunk
    mesh = plsc.VectorSubcoreMesh(core_axis_name="c", subcore_axis_name="s")

    @functools.partial(
        pl.kernel,
        out_type=jax.ShapeDtypeStruct((total_rows, N), jnp.float32),
        mesh=mesh,
        scratch_types=[
            pltpu.VMEM((chunk,), jnp.int32),
            pltpu.VMEM((chunk, N), jnp.float32),
            pltpu.SemaphoreType.DMA,
        ],
        compiler_params=pltpu.CompilerParams(use_tc_tiling_on_sc=False),
    )
    def gather_kernel(table_hbm, idx_hbm, out_hbm, idx_v, rows_v, sem):
        wid = lax.axis_index("s") * nc + lax.axis_index("c")
        base = wid * b_per_w

        def body(j, carry):
            off = base + j * chunk
            pltpu.sync_copy(idx_hbm.at[pl.ds(off, chunk)], idx_v)
            copies = []
            for k in range(n_sub):
                copies.append(
                    pltpu.async_copy(
                        table_hbm.at[idx_v.at[pl.ds(k * sub, sub)]],
                        rows_v.at[pl.ds(k * sub, sub)],
                        sem,
                    )
                )
            for c in copies:
                c.wait()
            pltpu.sync_copy(rows_v, out_hbm.at[pl.ds(off, chunk)])
            return carry

        lax.fori_loop(0, n_chunks, body, 0)

    return gather_kernel(table, idx_flat)


def kernel(inputs, mapping_array):
    batch, feat = inputs.shape
    num_ccsk = feat // NUM_BITS

    shifts = (2 ** jnp.arange(NUM_BITS - 1, -1, -1, dtype=jnp.float32))
    w = jnp.zeros((feat, num_ccsk), jnp.float32)
    cols = jnp.repeat(jnp.arange(num_ccsk), NUM_BITS)
    rows = jnp.arange(feat)
    w = w.at[rows, cols].set(jnp.tile(shifts, num_ccsk))

    idx = _pack_indices(inputs, w, block_rows=512)
    idx_flat = idx.reshape(batch * num_ccsk)
    out = _gather_rows(mapping_array, idx_flat, batch * num_ccsk)
    return out.reshape(batch, num_ccsk * N)
